# Initial kernel scaffold; baseline (speedup 1.0000x reference)
import jax
import jax.numpy as jnp
from jax.experimental import pallas as pl


def kernel(node_data, node_num, lap_eigvec, lap_eigval, edge_index, edge_data, edge_num, W_emb, W_lap, order_table):
    raise NotImplementedError("write your pallas kernel here")


# speedup vs baseline: 2.1966x; 2.1966x over previous
import functools

import jax
import jax.numpy as jnp
from jax import lax
from jax.experimental import pallas as pl
from jax.experimental.pallas import tpu as pltpu
from jax.experimental.pallas import tpu_sc as plsc


def _prep_body(K, BN, N, ev_ref, wl_ref, ot_ref, src_ref, dst_ref,
               aux_ref, srcg_ref, dstg_ref):
    ev = ev_ref[...]
    wl = wl_ref[...]
    half = 0.5 * ot_ref[0:1, :]
    dn = (((1,), (1,)), ((), ()))
    a1 = lax.dot_general(ev, wl[:, :K], dn, preferred_element_type=jnp.float32)
    a2 = lax.dot_general(ev, wl[:, K:], dn, preferred_element_type=jnp.float32)
    a1 = a1 + half
    a2 = a2 + half
    a3 = a2 + (ot_ref[1:2, :] - ot_ref[0:1, :])
    aux_ref[0:BN, :] = a1
    aux_ref[BN:2 * BN, :] = a2
    aux_ref[2 * BN:3 * BN, :] = a3
    s = src_ref[...]
    d = dst_ref[...]
    boff = lax.broadcasted_iota(jnp.int32, s.shape, 0) * N
    srcg_ref[...] = boff + s
    dstg_ref[...] = BN + boff + d + jnp.where(s == d, BN, 0).astype(jnp.int32)


def _make_sc_gather_sum(BT, D, F, V, AUXR):
    info = plsc.get_sparse_core_info()
    NC, NS, L = info.num_cores, info.num_subcores, info.num_lanes
    NW = NC * NS
    TW = BT // NW
    C = 8
    n_chunks = TW // C
    mesh = plsc.VectorSubcoreMesh(core_axis_name="c", subcore_axis_name="s")

    @functools.partial(
        pl.kernel, mesh=mesh,
        out_type=jax.ShapeDtypeStruct((BT, D), jnp.float32),
        scratch_types=[
            pltpu.VMEM((C * F,), jnp.int32),
            pltpu.VMEM((C * 2,), jnp.int32),
            pltpu.VMEM((C * F, D), jnp.float32),
            pltpu.VMEM((C * 2, D), jnp.float32),
            pltpu.VMEM((C, D), jnp.float32),
            pltpu.SemaphoreType.DMA,
        ],
    )
    def sc_kernel(emb_hbm, aux_hbm, eidx_hbm, aidx_hbm, out_hbm,
                  eidx_v, aidx_v, erows_v, arows_v, outc_v, sem):
        wid = lax.axis_index("s") * NC + lax.axis_index("c")
        base_tok = wid * TW

        def chunk_body(g, carry):
            tok0 = pl.multiple_of(base_tok + g * C, C)
            pltpu.sync_copy(eidx_hbm.at[pl.ds(tok0 * F, C * F)], eidx_v)
            pltpu.sync_copy(aidx_hbm.at[pl.ds(tok0 * 2, C * 2)], aidx_v)
            cp1 = pltpu.async_copy(emb_hbm.at[eidx_v], erows_v, sem)
            cp2 = pltpu.async_copy(aux_hbm.at[aidx_v], arows_v, sem)
            cp1.wait()
            cp2.wait()

            def tok_body(c, carry2):
                for dsl in range(D // L):
                    sl = pl.ds(dsl * L, L)
                    acc = erows_v[c * F, sl]
                    for f in range(1, F):
                        acc = acc + erows_v[c * F + f, sl]
                    acc = acc + arows_v[2 * c, sl] + arows_v[2 * c + 1, sl]
                    outc_v[c, sl] = acc
                return carry2

            lax.fori_loop(0, C, tok_body, 0)
            pltpu.sync_copy(outc_v, out_hbm.at[pl.ds(tok0, C)])
            return carry

        lax.fori_loop(0, n_chunks, chunk_body, 0)

    return sc_kernel


def kernel(node_data, node_num, lap_eigvec, lap_eigval, edge_index,
           edge_data, edge_num, W_emb, W_lap, order_table):
    B, N, F = node_data.shape
    E = edge_data.shape[1]
    K = lap_eigvec.shape[1]
    V, D = W_emb.shape
    T = N + E
    BN = B * N
    BT = B * T

    iota_n = jnp.broadcast_to(jnp.arange(N, dtype=edge_index.dtype)[None, :], (B, N))
    eidx2 = edge_index.reshape(2, B, E)
    src = jnp.concatenate([iota_n, eidx2[0]], axis=1)
    dst = jnp.concatenate([iota_n, eidx2[1]], axis=1)
    padded_index = jnp.stack([src, dst], axis=-1)
    emb_idx = jnp.concatenate([node_data, edge_data], axis=1).reshape(BT * F)

    token_pos = jnp.broadcast_to(jnp.arange(T, dtype=jnp.int32)[None, :], (B, T))
    seq_len = (node_num + edge_num).astype(jnp.int32)[:, None]
    padding_mask = token_pos >= seq_len

    aux_table, srcg, dstg = pl.pallas_call(
        functools.partial(_prep_body, K, BN, N),
        out_shape=[
            jax.ShapeDtypeStruct((3 * BN, D), jnp.float32),
            jax.ShapeDtypeStruct((B, T), jnp.int32),
            jax.ShapeDtypeStruct((B, T), jnp.int32),
        ],
    )(lap_eigvec, W_lap, order_table, src.astype(jnp.int32), dst.astype(jnp.int32))
    aux_idx = jnp.stack([srcg, dstg], axis=-1).reshape(BT * 2)

    sc = _make_sc_gather_sum(BT, D, F, V, 3 * BN)
    out = sc(W_emb, aux_table, emb_idx, aux_idx)
    padded_feature = out.reshape(B, T, D)
    return (padded_feature, padding_mask, padded_index)


# speedup vs baseline: 3.5835x; 1.6314x over previous
import functools

import jax
import jax.numpy as jnp
from jax import lax
from jax.experimental import pallas as pl
from jax.experimental.pallas import tpu as pltpu
from jax.experimental.pallas import tpu_sc as plsc


def _prep_body(K, BN, N, ev_ref, wl_ref, ot_ref, src_ref, dst_ref,
               aux_ref, srcg_ref, dstg_ref):
    ev = ev_ref[...]
    wl = wl_ref[...]
    half = 0.5 * ot_ref[0:1, :]
    dn = (((1,), (1,)), ((), ()))
    a1 = lax.dot_general(ev, wl[:, :K], dn, preferred_element_type=jnp.float32)
    a2 = lax.dot_general(ev, wl[:, K:], dn, preferred_element_type=jnp.float32)
    a1 = a1 + half
    a2 = a2 + half
    a3 = a2 + (ot_ref[1:2, :] - ot_ref[0:1, :])
    aux_ref[0:BN, :] = a1
    aux_ref[BN:2 * BN, :] = a2
    aux_ref[2 * BN:3 * BN, :] = a3
    s = src_ref[...]
    d = dst_ref[...]
    boff = lax.broadcasted_iota(jnp.int32, s.shape, 0) * N
    srcg_ref[...] = boff + s
    dstg_ref[...] = BN + boff + d + jnp.where(s == d, BN, 0).astype(jnp.int32)


def _tree_sum(vals):
    while len(vals) > 1:
        nxt = [vals[i] + vals[i + 1] for i in range(0, len(vals) - 1, 2)]
        if len(vals) % 2:
            nxt.append(vals[-1])
        vals = nxt
    return vals[0]


def _make_sc_gather_sum(BT, D, F):
    info = plsc.get_sparse_core_info()
    NC, NS, L = info.num_cores, info.num_subcores, info.num_lanes
    NW = NC * NS
    TW = BT // NW
    C = 8
    H = C // 2
    G = C * (F + 2)
    n_chunks = TW // C
    NSL = D // L
    mesh = plsc.VectorSubcoreMesh(core_axis_name="c", subcore_axis_name="s")

    @functools.partial(
        pl.kernel, mesh=mesh,
        out_type=jax.ShapeDtypeStruct((BT, D), jnp.float32),
        scratch_types=[
            pltpu.VMEM((2, G), jnp.int32),
            pltpu.VMEM((2, C * F, D), jnp.float32),
            pltpu.VMEM((2, C * 2, D), jnp.float32),
            pltpu.VMEM((2, H, D), jnp.float32),
            pltpu.SemaphoreType.DMA,
            pltpu.SemaphoreType.DMA,
            pltpu.SemaphoreType.DMA,
        ],
    )
    def sc_kernel(emb_hbm, aux_hbm, cidx_hbm, out_hbm,
                  cidx_v, erows_v, arows_v, outh_v, sem_i, sem_r, sem_s):
        wid = lax.axis_index("s") * NC + lax.axis_index("c")
        base_tok = pl.multiple_of(wid * TW, TW)
        base_chunk = wid * n_chunks

        def idx_src(g):
            return cidx_hbm.at[pl.ds(pl.multiple_of((base_chunk + g) * G, G), G)]

        def fire_idx(g, p):
            pltpu.async_copy(idx_src(g), cidx_v.at[p], sem_i)

        def fire_gathers(p):
            pltpu.async_copy(emb_hbm.at[cidx_v.at[p, pl.ds(0, C * F)]],
                             erows_v.at[p], sem_r)
            pltpu.async_copy(aux_hbm.at[cidx_v.at[p, pl.ds(C * F, C * 2)]],
                             arows_v.at[p], sem_r)

        def wait_idx(g, p):
            pltpu.make_async_copy(idx_src(g), cidx_v.at[p], sem_i).wait()

        def wait_gathers(p):
            pltpu.make_async_copy(emb_hbm.at[cidx_v.at[p, pl.ds(0, C * F)]],
                                  erows_v.at[p], sem_r).wait()
            pltpu.make_async_copy(aux_hbm.at[cidx_v.at[p, pl.ds(C * F, C * 2)]],
                                  arows_v.at[p], sem_r).wait()

        def wait_store(h):
            pltpu.make_async_copy(outh_v.at[h], out_hbm.at[pl.ds(0, H)],
                                  sem_s).wait()

        pltpu.sync_copy(idx_src(0), cidx_v.at[0])
        fire_gathers(0)
        fire_idx(1, 1)

        def outer(i, carry):
            for sub in (0, 1):
                g = 2 * i + sub
                q = 1 - sub
                if sub == 0:
                    wait_idx(g + 1, q)
                    fire_gathers(q)
                else:
                    @pl.when(i < (n_chunks // 2) - 1)
                    def _():
                        wait_idx(g + 1, q)
                        fire_gathers(q)
                wait_gathers(sub)
                @pl.when(g < n_chunks - 2)
                def _():
                    fire_idx(g + 2, sub)
                tok0 = base_tok + g * C
                for h in (0, 1):
                    @pl.when(g > 0)
                    def _():
                        wait_store(h)

                    def tok_body(c, carry2, sub=sub, h=h):
                        r = h * H + c
                        for dsl in range(NSL):
                            sl = pl.ds(dsl * L, L)
                            vals = [erows_v[sub, r * F + f, sl]
                                    for f in range(F)]
                            vals.append(arows_v[sub, 2 * r, sl])
                            vals.append(arows_v[sub, 2 * r + 1, sl])
                            outh_v[h, c, sl] = _tree_sum(vals)
                        return carry2

                    lax.fori_loop(0, H, tok_body, 0)
                    pltpu.async_copy(outh_v.at[h],
                                     out_hbm.at[pl.ds(tok0 + h * H, H)],
                                     sem_s)
            return carry

        lax.fori_loop(0, n_chunks // 2, outer, 0)
        wait_store(0)
        wait_store(1)

    return sc_kernel


def kernel(node_data, node_num, lap_eigvec, lap_eigval, edge_index,
           edge_data, edge_num, W_emb, W_lap, order_table):
    B, N, F = node_data.shape
    E = edge_data.shape[1]
    K = lap_eigvec.shape[1]
    V, D = W_emb.shape
    T = N + E
    BN = B * N
    BT = B * T

    iota_n = jnp.broadcast_to(jnp.arange(N, dtype=edge_index.dtype)[None, :], (B, N))
    eidx2 = edge_index.reshape(2, B, E)
    src = jnp.concatenate([iota_n, eidx2[0]], axis=1)
    dst = jnp.concatenate([iota_n, eidx2[1]], axis=1)
    padded_index = jnp.stack([src, dst], axis=-1)
    emb_idx = jnp.concatenate([node_data, edge_data], axis=1).reshape(BT, F)

    token_pos = jnp.broadcast_to(jnp.arange(T, dtype=jnp.int32)[None, :], (B, T))
    seq_len = (node_num + edge_num).astype(jnp.int32)[:, None]
    padding_mask = token_pos >= seq_len

    aux_table, srcg, dstg = pl.pallas_call(
        functools.partial(_prep_body, K, BN, N),
        out_shape=[
            jax.ShapeDtypeStruct((3 * BN, D), jnp.float32),
            jax.ShapeDtypeStruct((B, T), jnp.int32),
            jax.ShapeDtypeStruct((B, T), jnp.int32),
        ],
    )(lap_eigvec, W_lap, order_table, src.astype(jnp.int32), dst.astype(jnp.int32))

    C = 8
    embc = emb_idx.astype(jnp.int32).reshape(BT // C, C * F)
    auxc = jnp.stack([srcg, dstg], axis=-1).reshape(BT // C, C * 2)
    cidx = jnp.concatenate([embc, auxc], axis=1).reshape(BT * (F + 2))

    sc = _make_sc_gather_sum(BT, D, F)
    out = sc(W_emb, aux_table, cidx)
    padded_feature = out.reshape(B, T, D)
    return (padded_feature, padding_mask, padded_index)
